# baseline (speedup 1.0000x reference)
import functools

import jax
import jax.numpy as jnp
from jax.experimental import pallas as pl
from jax.experimental.pallas import tpu as pltpu
from jax.experimental.pallas import tpu_sc as plsc

NUM_TOKENS = 8192
CODE_DIM = 32
BETA = 0.25

B_ROWS = 512
N_ROWS = 8 * 32 * 32
NB = N_ROWS // B_ROWS
NCORES = 2
NB_IN = NB // NCORES
WIN = 2048

GATHER_WIN = 256


def _vq_kernel(z_ref, e_ref, zsq_ref, esq_ref, enc_ref, idx_ref, counts_ref,
               sqerr_ref, acc_counts, acc_sqerr):
    ii = pl.program_id(1)
    zb = z_ref[...]
    e = e_ref[...]

    @pl.when(ii == 0)
    def _init():
        acc_counts[...] = jnp.zeros_like(acc_counts)
        acc_sqerr[0] = 0.0

    prod = jax.lax.dot_general(zb.astype(jnp.bfloat16), e.astype(jnp.bfloat16),
                               (((1,), (1,)), ((), ())),
                               preferred_element_type=jnp.float32)
    d = zsq_ref[...] + esq_ref[...] - 2.0 * prod

    BIG = jnp.int32(2 ** 30)
    gv = jnp.full((B_ROWS, 1), jnp.inf, jnp.float32)
    gr = jnp.full((B_ROWS, 1), jnp.inf, jnp.float32)
    gi = jnp.zeros((B_ROWS, 1), jnp.int32)
    for w in range(NUM_TOKENS // WIN):
        dw = d[:, w * WIN:(w + 1) * WIN]
        mw = jnp.min(dw, axis=1, keepdims=True)
        iota_w = (jax.lax.broadcasted_iota(jnp.int32, (B_ROWS, WIN), 1)
                  + w * WIN)
        iw = jnp.min(jnp.where(dw == mw, iota_w, BIG), axis=1, keepdims=True)
        repl = mw < gv
        gi = jnp.where(repl, iw, gi)
        gr = jnp.where(repl, mw, gr)
        gv = jnp.where(repl, mw.astype(jnp.bfloat16).astype(jnp.float32), gv)

    enc = (jax.lax.broadcasted_iota(jnp.int32, (B_ROWS, NUM_TOKENS), 1)
           == gi).astype(jnp.float32)
    enc_ref[...] = enc
    idx_ref[0, 0] = gi[:, 0]

    ones_row = jnp.ones((1, B_ROWS), jnp.float32)
    acc_counts[...] += jax.lax.dot_general(
        ones_row, enc, (((1,), (0,)), ((), ())),
        preferred_element_type=jnp.float32)
    acc_sqerr[0] += jnp.sum(jnp.maximum(gr, 0.0))

    @pl.when(ii == NB_IN - 1)
    def _finish():
        counts_ref[0] = acc_counts[...]
        sqerr_ref[0] = jnp.full((1, 1), acc_sqerr[0], jnp.float32)


_SC_WORKERS = 32
_B_PER_W = N_ROWS // _SC_WORKERS


def _sc_gather(e, idx_flat):
    mesh = plsc.VectorSubcoreMesh(core_axis_name="c", subcore_axis_name="s")

    @functools.partial(
        pl.kernel, mesh=mesh,
        out_type=jax.ShapeDtypeStruct((N_ROWS, 128), jnp.float32),
        scratch_types=[
            pltpu.VMEM((_B_PER_W,), jnp.int32),
            pltpu.VMEM((_B_PER_W, 128), jnp.float32),
            pltpu.SemaphoreType.DMA,
        ],
    )
    def gather_kernel(e_hbm, idx_hbm, out_hbm, idx_v, rows_v, sem):
        wid = jax.lax.axis_index("s") * 2 + jax.lax.axis_index("c")
        base = wid * _B_PER_W
        pltpu.sync_copy(idx_hbm.at[pl.ds(base, _B_PER_W)], idx_v)
        pltpu.async_copy(e_hbm.at[idx_v], rows_v, sem).wait()
        pltpu.sync_copy(rows_v, out_hbm.at[pl.ds(base, _B_PER_W)])

    return gather_kernel(e, idx_flat)


@jax.jit
def kernel(z, embedding_weight):
    zt = jnp.transpose(z, (0, 2, 3, 1))
    z_flat = zt.reshape(N_ROWS, CODE_DIM)
    z_sq_in = jnp.sum(z_flat ** 2, axis=1, keepdims=True)
    e_sq_in = jnp.sum(embedding_weight ** 2, axis=1)[None, :]
    enc, idx3, counts2, sqerr2 = pl.pallas_call(
        _vq_kernel,
        grid=(NCORES, NB_IN),
        in_specs=[
            pl.BlockSpec((B_ROWS, CODE_DIM),
                         lambda o, i: (o * NB_IN + i, 0)),
            pl.BlockSpec((NUM_TOKENS, CODE_DIM), lambda o, i: (0, 0)),
            pl.BlockSpec((B_ROWS, 1), lambda o, i: (o * NB_IN + i, 0)),
            pl.BlockSpec((1, NUM_TOKENS), lambda o, i: (0, 0)),
        ],
        out_specs=[
            pl.BlockSpec((B_ROWS, NUM_TOKENS), lambda o, i: (o * NB_IN + i, 0)),
            pl.BlockSpec((1, 1, B_ROWS), lambda o, i: (o * NB_IN + i, 0, 0)),
            pl.BlockSpec((1, 1, NUM_TOKENS), lambda o, i: (o, 0, 0)),
            pl.BlockSpec((1, 1, 1), lambda o, i: (o, 0, 0)),
        ],
        out_shape=[
            jax.ShapeDtypeStruct((N_ROWS, NUM_TOKENS), jnp.float32),
            jax.ShapeDtypeStruct((NB, 1, B_ROWS), jnp.int32),
            jax.ShapeDtypeStruct((NCORES, 1, NUM_TOKENS), jnp.float32),
            jax.ShapeDtypeStruct((NCORES, 1, 1), jnp.float32),
        ],
        scratch_shapes=[
            pltpu.VMEM((1, NUM_TOKENS), jnp.float32),
            pltpu.SMEM((1,), jnp.float32),
        ],
        compiler_params=pltpu.CompilerParams(
            dimension_semantics=("parallel", "arbitrary")),
    )(z_flat, embedding_weight, z_sq_in, e_sq_in)

    encoding_indices = idx3.reshape(N_ROWS)
    e_pad = jnp.pad(embedding_weight, ((0, 0), (0, 128 - CODE_DIM)))
    zq_flat = _sc_gather(e_pad, encoding_indices)[:, :CODE_DIM]

    counts = counts2[:, 0, :].sum(axis=0)
    mse = sqerr2.sum() / float(N_ROWS * CODE_DIM)
    loss = BETA * mse + mse
    probs = counts / float(N_ROWS)
    perplexity = jnp.exp(-jnp.sum(probs * jnp.log(probs + 1e-10)))
    zq_out = jnp.transpose(zq_flat.reshape(8, 32, 32, CODE_DIM), (0, 3, 1, 2))
    return (zq_out, loss, perplexity, enc, encoding_indices)
